# baseline (speedup 1.0000x reference)
import jax
import jax.numpy as jnp
from jax.experimental import pallas as pl
from jax.experimental.pallas import tpu as pltpu

IN_FEATURES = 3
H1 = 64
H2 = 32
NUM_CLASSES = 5

PACK = 8
GROUPS = 16
RECS_PER_ROW = PACK * GROUPS
ROW_TILE = 1000

BATCH_TILE = 4096


def _round_up(n, m):
    return m * pl.cdiv(n, m)


def _block_diag(w, p):
    k, n = w.shape
    eye = jnp.eye(p, dtype=w.dtype)
    return (eye[:, None, :, None] * w[None, :, None, :]).reshape(p * k, p * n)


def _wide_mlp_kernel(x_ref, w1_ref, b1_ref, w2_ref, b2_ref, w3_ref, b3_ref,
                     o_ref):
    x = x_ref[...]
    w1 = w1_ref[...]
    b1 = b1_ref[...]
    w2 = w2_ref[...]
    b2 = b2_ref[...]
    w3 = w3_ref[...]
    b3 = b3_ref[...]
    ki = IN_FEATURES * PACK
    ko = NUM_CLASSES * PACK
    for g in range(GROUPS):
        xg = x[:, g * ki:(g + 1) * ki]
        h1 = jnp.dot(xg, w1, preferred_element_type=jnp.float32)
        h1 = jnp.maximum(h1 + b1, 0.0)
        h2 = jnp.dot(h1, w2, preferred_element_type=jnp.float32)
        h2 = jnp.maximum(h2 + b2, 0.0)
        out = jnp.dot(h2, w3, preferred_element_type=jnp.float32)
        o_ref[:, g * ko:(g + 1) * ko] = (out + b3).astype(o_ref.dtype)


def _mlp_kernel(x_ref, w1_ref, b1_ref, w2_ref, b2_ref, w3_ref, b3_ref, o_ref):
    x = x_ref[...]
    h1 = jnp.dot(x, w1_ref[...], preferred_element_type=jnp.float32)
    h1 = jnp.maximum(h1 + b1_ref[...], 0.0)
    h2 = jnp.dot(h1, w2_ref[...], preferred_element_type=jnp.float32)
    h2 = jnp.maximum(h2 + b2_ref[...], 0.0)
    out = jnp.dot(h2, w3_ref[...], preferred_element_type=jnp.float32)
    o_ref[...] = (out + b3_ref[...]).astype(o_ref.dtype)


def _const_map(i):
    return (0, 0)


def _batch_map(i):
    return (i, 0)


def _wide_path(x, w1, b1, w2, b2, w3, b3):
    B = x.shape[0]
    rows = B // RECS_PER_ROW
    xw = x.reshape(rows, IN_FEATURES * RECS_PER_ROW)

    w1t = _block_diag(w1[:IN_FEATURES], PACK)
    w2t = _block_diag(w2, PACK)
    w3t = _block_diag(w3, PACK)
    b1t = jnp.tile(b1, (1, PACK))
    b2t = jnp.tile(b2, (1, PACK))
    b3t = jnp.tile(b3, (1, PACK))

    rt = min(ROW_TILE, _round_up(rows, 8))
    grid = (pl.cdiv(rows, rt),)
    in_w = IN_FEATURES * RECS_PER_ROW
    out_w = NUM_CLASSES * RECS_PER_ROW

    out = pl.pallas_call(
        _wide_mlp_kernel,
        out_shape=jax.ShapeDtypeStruct((rows, out_w), jnp.float32),
        grid=grid,
        in_specs=[
            pl.BlockSpec((rt, in_w), _batch_map),
            pl.BlockSpec((IN_FEATURES * PACK, H1 * PACK), _const_map),
            pl.BlockSpec((1, H1 * PACK), _const_map),
            pl.BlockSpec((H1 * PACK, H2 * PACK), _const_map),
            pl.BlockSpec((1, H2 * PACK), _const_map),
            pl.BlockSpec((H2 * PACK, NUM_CLASSES * PACK), _const_map),
            pl.BlockSpec((1, NUM_CLASSES * PACK), _const_map),
        ],
        out_specs=pl.BlockSpec((rt, out_w), _batch_map),
        compiler_params=pltpu.CompilerParams(
            dimension_semantics=("parallel",)),
    )(xw, w1t, b1t, w2t, b2t, w3t, b3t)

    return out.reshape(B, NUM_CLASSES)


def _direct_path(x, w1, b1, w2, b2, w3, b3):
    B = x.shape[0]
    w1c = w1[:IN_FEATURES]
    bt = min(BATCH_TILE, _round_up(B, 8))
    grid = (pl.cdiv(B, bt),)

    return pl.pallas_call(
        _mlp_kernel,
        out_shape=jax.ShapeDtypeStruct((B, NUM_CLASSES), jnp.float32),
        grid=grid,
        in_specs=[
            pl.BlockSpec((bt, IN_FEATURES), _batch_map),
            pl.BlockSpec((IN_FEATURES, H1), _const_map),
            pl.BlockSpec((1, H1), _const_map),
            pl.BlockSpec((H1, H2), _const_map),
            pl.BlockSpec((1, H2), _const_map),
            pl.BlockSpec((H2, NUM_CLASSES), _const_map),
            pl.BlockSpec((1, NUM_CLASSES), _const_map),
        ],
        out_specs=pl.BlockSpec((bt, NUM_CLASSES), _batch_map),
        compiler_params=pltpu.CompilerParams(
            dimension_semantics=("parallel",)),
    )(x, w1c, b1, w2, b2, w3, b3)


@jax.jit
def kernel(x, w1, b1, w2, b2, w3, b3):
    if x.shape[0] % RECS_PER_ROW == 0:
        return _wide_path(x, w1, b1, w2, b2, w3, b3)
    return _direct_path(x, w1, b1, w2, b2, w3, b3)


# speedup vs baseline: 57.8659x; 57.8659x over previous
import jax
import jax.numpy as jnp
from jax.experimental import pallas as pl
from jax.experimental.pallas import tpu as pltpu

IN_FEATURES = 3
H1 = 64
H2 = 32
NUM_CLASSES = 5

LANE_TILE = 32768


def _round_up(n, m):
    return m * pl.cdiv(n, m)


def _tmlp_kernel(x_ref, w1_ref, b1_ref, w2_ref, b2_ref, w3_ref, b3_ref,
                 o_ref):
    x = x_ref[...]
    h1 = jnp.dot(w1_ref[...], x, preferred_element_type=jnp.float32)
    h1 = jnp.maximum(h1 + b1_ref[...], 0.0)
    h2 = jnp.dot(w2_ref[...], h1, preferred_element_type=jnp.float32)
    h2 = jnp.maximum(h2 + b2_ref[...], 0.0)
    out = jnp.dot(w3_ref[...], h2, preferred_element_type=jnp.float32)
    o_ref[...] = (out + b3_ref[...]).astype(o_ref.dtype)


@jax.jit
def kernel(x, w1, b1, w2, b2, w3, b3):
    B = x.shape[0]

    xt = x.T
    w1t = w1[:IN_FEATURES].T
    w2t = w2.T
    w3t = w3.T
    b1t = b1.T
    b2t = b2.T
    b3t = b3.T

    bt = min(LANE_TILE, _round_up(B, 128))
    grid = (pl.cdiv(B, bt),)

    def lane_map(i):
        return (0, i)

    def const_map(i):
        return (0, 0)

    out = pl.pallas_call(
        _tmlp_kernel,
        out_shape=jax.ShapeDtypeStruct((NUM_CLASSES, B), jnp.float32),
        grid=grid,
        in_specs=[
            pl.BlockSpec((IN_FEATURES, bt), lane_map),
            pl.BlockSpec((H1, IN_FEATURES), const_map),
            pl.BlockSpec((H1, 1), const_map),
            pl.BlockSpec((H2, H1), const_map),
            pl.BlockSpec((H2, 1), const_map),
            pl.BlockSpec((NUM_CLASSES, H2), const_map),
            pl.BlockSpec((NUM_CLASSES, 1), const_map),
        ],
        out_specs=pl.BlockSpec((NUM_CLASSES, bt), lane_map),
        compiler_params=pltpu.CompilerParams(
            dimension_semantics=("parallel",)),
    )(xt, w1t, b1t, w2t, b2t, w3t, b3t)

    return out.T
